# Initial kernel scaffold; baseline (speedup 1.0000x reference)
import jax
import jax.numpy as jnp
from jax.experimental import pallas as pl


def kernel(rel_pos_matrix, len, embeddings_table):
    raise NotImplementedError("write your pallas kernel here")


# speedup vs baseline: 5.2033x; 5.2033x over previous
import functools

import jax
import jax.numpy as jnp
from jax import lax
from jax.experimental import pallas as pl
from jax.experimental.pallas import tpu as pltpu
from jax.experimental.pallas import tpu_sc as plsc

_NC = 2
_NS = 16
_NW = _NC * _NS
_LANES = 16

_CHUNK = 1024
_SUB = 128


def _sc_gather(args, B, D, n_chunks):
  per_w = B // _NW
  mesh = plsc.VectorSubcoreMesh(core_axis_name="c", subcore_axis_name="s")

  @functools.partial(
      pl.kernel,
      out_type=jax.ShapeDtypeStruct((B, D), jnp.float32),
      mesh=mesh,
      scratch_types=[
          pltpu.VMEM((_CHUNK,), jnp.int32),
          pltpu.VMEM((_CHUNK, D), jnp.float32),
          pltpu.VMEM((_LANES,), jnp.int32),
          pltpu.SemaphoreType.DMA,
      ],
      compiler_params=pltpu.CompilerParams(use_tc_tiling_on_sc=False),
  )
  def k(idx_hbm, off_hbm, table_hbm, out_hbm, idx_v, rows_v, off_v, sem):
    wid = lax.axis_index("s") * _NC + lax.axis_index("c")
    base = wid * per_w
    pltpu.sync_copy(off_hbm, off_v)
    offv = off_v[...]

    def group(g, carry):
      start = base + g * _CHUNK
      pltpu.sync_copy(idx_hbm.at[pl.ds(start, _CHUNK)], idx_v)
      for i in range(_CHUNK // _LANES):
        sl = pl.ds(i * _LANES, _LANES)
        idx_v[sl] = idx_v[sl] + offv
      cps = [
          pltpu.async_copy(
              table_hbm.at[idx_v.at[pl.ds(j * _SUB, _SUB)]],
              rows_v.at[pl.ds(j * _SUB, _SUB), :],
              sem,
          )
          for j in range(_CHUNK // _SUB)
      ]
      for cp in cps:
        cp.wait()
      pltpu.sync_copy(rows_v, out_hbm.at[pl.ds(start, _CHUNK), :])
      return carry

    lax.fori_loop(0, n_chunks, group, None)

  idx, off_vec, table = args
  return k(idx, off_vec, table)


def kernel(rel_pos_matrix, len, embeddings_table):
  n = rel_pos_matrix.shape[0]
  D = embeddings_table.shape[1]
  B = n * n
  idx = rel_pos_matrix.reshape(B)
  off = jnp.asarray(len, jnp.int32) - jnp.int32(n)
  off_vec = jnp.full((_LANES,), off, dtype=jnp.int32)
  n_chunks = per_w_chunks(B)
  out = _sc_gather((idx, off_vec, embeddings_table), B, D, n_chunks)
  return out.reshape(n, n, D)


def per_w_chunks(B):
  per_w = B // _NW
  assert per_w % _CHUNK == 0, (B, per_w)
  return per_w // _CHUNK


# speedup vs baseline: 5.2229x; 1.0038x over previous
import functools

import jax
import jax.numpy as jnp
from jax import lax
from jax.experimental import pallas as pl
from jax.experimental.pallas import tpu as pltpu
from jax.experimental.pallas import tpu_sc as plsc

_NC = 2
_NS = 16
_NW = _NC * _NS
_LANES = 16

_CHUNK = 1024
_SUB = 128


def _sc_gather(args, B, D, n_groups):
  per_w = B // _NW
  assert n_groups % 2 == 0 and n_groups >= 4
  mesh = plsc.VectorSubcoreMesh(core_axis_name="c", subcore_axis_name="s")

  @functools.partial(
      pl.kernel,
      out_type=jax.ShapeDtypeStruct((B, D), jnp.float32),
      mesh=mesh,
      scratch_types=[
          pltpu.VMEM((2, _CHUNK), jnp.int32),
          pltpu.VMEM((2, _CHUNK, D), jnp.float32),
          pltpu.VMEM((_LANES,), jnp.int32),
          pltpu.SemaphoreType.DMA,
          pltpu.SemaphoreType.DMA,
          pltpu.SemaphoreType.DMA,
          pltpu.SemaphoreType.DMA,
      ],
      compiler_params=pltpu.CompilerParams(use_tc_tiling_on_sc=False),
  )
  def k(idx_hbm, off_hbm, table_hbm, out_hbm, idx_v, rows_v, off_v, sem_i,
        sem_g, sem_o0, sem_o1):
    wid = lax.axis_index("s") * _NC + lax.axis_index("c")
    base = wid * per_w
    sem_o = (sem_o0, sem_o1)
    pltpu.sync_copy(off_hbm, off_v)
    offv = off_v[...]

    def add_off(p):
      iv = idx_v.at[p]
      for i in range(_CHUNK // _LANES):
        sl = pl.ds(i * _LANES, _LANES)
        iv[sl] = iv[sl] + offv

    pltpu.sync_copy(idx_hbm.at[pl.ds(base, _CHUNK)], idx_v.at[0])
    add_off(0)
    pltpu.async_copy(idx_hbm.at[pl.ds(base + _CHUNK, _CHUNK)], idx_v.at[1],
                     sem_i)

    def half(g2, p):
      g = g2 * 2 + p
      start = base + g * _CHUNK
      iv = idx_v.at[p]
      rv = rows_v.at[p]

      @pl.when(g2 >= 1)
      def _():
        pltpu.make_async_copy(rv, out_hbm.at[pl.ds(start, _CHUNK), :],
                              sem_o[p]).wait()

      gathers = [
          pltpu.async_copy(
              table_hbm.at[iv.at[pl.ds(j * _SUB, _SUB)]],
              rv.at[pl.ds(j * _SUB, _SUB), :],
              sem_g,
          )
          for j in range(_CHUNK // _SUB)
      ]

      def stage_next():
        pltpu.make_async_copy(idx_hbm.at[pl.ds(start, _CHUNK)],
                              idx_v.at[1 - p], sem_i).wait()
        add_off(1 - p)
      if p == 0:
        stage_next()
      else:
        pl.when(g2 < n_groups // 2 - 1)(stage_next)

      for cp in gathers:
        cp.wait()

      @pl.when(g2 < n_groups // 2 - 1)
      def _():
        pltpu.async_copy(idx_hbm.at[pl.ds(start + 2 * _CHUNK, _CHUNK)], iv,
                         sem_i)

      pltpu.async_copy(rv, out_hbm.at[pl.ds(start, _CHUNK), :], sem_o[p])

    def pair(g2, carry):
      half(g2, 0)
      half(g2, 1)
      return carry

    lax.fori_loop(0, n_groups // 2, pair, None)

    tail = base + (n_groups - 2) * _CHUNK
    pltpu.make_async_copy(rows_v.at[0], out_hbm.at[pl.ds(tail, _CHUNK), :],
                          sem_o[0]).wait()
    pltpu.make_async_copy(rows_v.at[1],
                          out_hbm.at[pl.ds(tail + _CHUNK, _CHUNK), :],
                          sem_o[1]).wait()

  idx, off_vec, table = args
  return k(idx, off_vec, table)


def kernel(rel_pos_matrix, len, embeddings_table):
  n = rel_pos_matrix.shape[0]
  D = embeddings_table.shape[1]
  B = n * n
  idx = rel_pos_matrix.reshape(B)
  off = jnp.asarray(len, jnp.int32) - jnp.int32(n)
  off_vec = jnp.full((_LANES,), off, dtype=jnp.int32)
  per_w = B // _NW
  assert per_w % _CHUNK == 0
  out = _sc_gather((idx, off_vec, embeddings_table), B, D, per_w // _CHUNK)
  return out.reshape(n, n, D)


# speedup vs baseline: 5.2234x; 1.0001x over previous
import functools

import jax
import jax.numpy as jnp
from jax import lax
from jax.experimental import pallas as pl
from jax.experimental.pallas import tpu as pltpu
from jax.experimental.pallas import tpu_sc as plsc

_NC = 2
_NS = 16
_NW = _NC * _NS
_LANES = 16

_CHUNK = 1024
_SUB = 512


def _sc_gather(args, B, D, n_groups):
  per_w = B // _NW
  assert n_groups % 2 == 0 and n_groups >= 4
  mesh = plsc.VectorSubcoreMesh(core_axis_name="c", subcore_axis_name="s")

  @functools.partial(
      pl.kernel,
      out_type=jax.ShapeDtypeStruct((B, D), jnp.float32),
      mesh=mesh,
      scratch_types=[
          pltpu.VMEM((2, _CHUNK), jnp.int32),
          pltpu.VMEM((2, _CHUNK, D), jnp.float32),
          pltpu.VMEM((_LANES,), jnp.int32),
          pltpu.SemaphoreType.DMA,
          pltpu.SemaphoreType.DMA,
          pltpu.SemaphoreType.DMA,
          pltpu.SemaphoreType.DMA,
      ],
      compiler_params=pltpu.CompilerParams(use_tc_tiling_on_sc=False),
  )
  def k(idx_hbm, off_hbm, table_hbm, out_hbm, idx_v, rows_v, off_v, sem_i,
        sem_g, sem_o0, sem_o1):
    wid = lax.axis_index("s") * _NC + lax.axis_index("c")
    base = wid * per_w
    sem_o = (sem_o0, sem_o1)
    pltpu.sync_copy(off_hbm, off_v)
    offv = off_v[...]

    def add_off(p):
      iv = idx_v.at[p]
      for i in range(_CHUNK // _LANES):
        sl = pl.ds(i * _LANES, _LANES)
        iv[sl] = iv[sl] + offv

    pltpu.sync_copy(idx_hbm.at[pl.ds(base, _CHUNK)], idx_v.at[0])
    add_off(0)
    pltpu.async_copy(idx_hbm.at[pl.ds(base + _CHUNK, _CHUNK)], idx_v.at[1],
                     sem_i)

    def half(g2, p):
      g = g2 * 2 + p
      start = base + g * _CHUNK
      iv = idx_v.at[p]
      rv = rows_v.at[p]

      @pl.when(g2 >= 1)
      def _():
        pltpu.make_async_copy(rv, out_hbm.at[pl.ds(start, _CHUNK), :],
                              sem_o[p]).wait()

      gathers = [
          pltpu.async_copy(
              table_hbm.at[iv.at[pl.ds(j * _SUB, _SUB)]],
              rv.at[pl.ds(j * _SUB, _SUB), :],
              sem_g,
          )
          for j in range(_CHUNK // _SUB)
      ]

      def stage_next():
        pltpu.make_async_copy(idx_hbm.at[pl.ds(start, _CHUNK)],
                              idx_v.at[1 - p], sem_i).wait()
        add_off(1 - p)
      if p == 0:
        stage_next()
      else:
        pl.when(g2 < n_groups // 2 - 1)(stage_next)

      for cp in gathers:
        cp.wait()

      @pl.when(g2 < n_groups // 2 - 1)
      def _():
        pltpu.async_copy(idx_hbm.at[pl.ds(start + 2 * _CHUNK, _CHUNK)], iv,
                         sem_i)

      pltpu.async_copy(rv, out_hbm.at[pl.ds(start, _CHUNK), :], sem_o[p])

    def pair(g2, carry):
      half(g2, 0)
      half(g2, 1)
      return carry

    lax.fori_loop(0, n_groups // 2, pair, None)

    tail = base + (n_groups - 2) * _CHUNK
    pltpu.make_async_copy(rows_v.at[0], out_hbm.at[pl.ds(tail, _CHUNK), :],
                          sem_o[0]).wait()
    pltpu.make_async_copy(rows_v.at[1],
                          out_hbm.at[pl.ds(tail + _CHUNK, _CHUNK), :],
                          sem_o[1]).wait()

  idx, off_vec, table = args
  return k(idx, off_vec, table)


def kernel(rel_pos_matrix, len, embeddings_table):
  n = rel_pos_matrix.shape[0]
  D = embeddings_table.shape[1]
  B = n * n
  idx = rel_pos_matrix.reshape(B)
  off = jnp.asarray(len, jnp.int32) - jnp.int32(n)
  off_vec = jnp.full((_LANES,), off, dtype=jnp.int32)
  per_w = B // _NW
  assert per_w % _CHUNK == 0
  out = _sc_gather((idx, off_vec, embeddings_table), B, D, per_w // _CHUNK)
  return out.reshape(n, n, D)
